# baseline (speedup 1.0000x reference)
import jax
import jax.numpy as jnp
from jax.experimental import pallas as pl
from jax.experimental.pallas import tpu as pltpu

D = 2048
E = 64
TOPK = 6
H = 1408
HS = 2816
T = 64

MGU = 704
NGU = 4
MDD = 1024
NDD = 2
NV = E + 2
RW = 5
RD = 4
JGU = NV * NGU
JD = NV * NDD


def _mega_kernel(x_ref, gatew_ref, srw_ref,
                 gw_hbm, uw_hbm, dw_hbm, sgw_hbm, suw_hbm, sdw_hbm,
                 out_ref,
                 ring_gu, ring_d, xt_ref, gt_ref, gut_ref, comb_ref, sg_ref,
                 sem_gu, sem_d):

    def issue_gu(j):
        @pl.when(j < JGU)
        def _():
            v = j // NGU
            m = j % NGU
            slot = j % RW
            row = (m % 2) * MGU

            def _expert():
                def _gate():
                    pltpu.make_async_copy(
                        gw_hbm.at[v, pl.ds(row, MGU), :],
                        ring_gu.at[slot], sem_gu.at[slot]).start()

                def _up():
                    pltpu.make_async_copy(
                        uw_hbm.at[v, pl.ds(row, MGU), :],
                        ring_gu.at[slot], sem_gu.at[slot]).start()
                jax.lax.cond(m < 2, _gate, _up)

            def _shared():
                srow = (v - E) * H + row

                def _gate():
                    pltpu.make_async_copy(
                        sgw_hbm.at[pl.ds(srow, MGU), :],
                        ring_gu.at[slot], sem_gu.at[slot]).start()

                def _up():
                    pltpu.make_async_copy(
                        suw_hbm.at[pl.ds(srow, MGU), :],
                        ring_gu.at[slot], sem_gu.at[slot]).start()
                jax.lax.cond(m < 2, _gate, _up)

            jax.lax.cond(v < E, _expert, _shared)

    def issue_d(k):
        @pl.when(k < JD)
        def _():
            v = k // NDD
            m = k % NDD
            slot = k % RD

            def _expert():
                pltpu.make_async_copy(
                    dw_hbm.at[v * NDD + m],
                    ring_d.at[slot], sem_d.at[slot]).start()

            def _shared():
                pltpu.make_async_copy(
                    sdw_hbm.at[pl.ds(m * MDD, MDD), pl.ds((v - E) * H, H)],
                    ring_d.at[slot], sem_d.at[slot]).start()

            jax.lax.cond(v < E, _expert, _shared)

    for j in range(RW - 1):
        issue_gu(jnp.int32(j))
    for k in range(RD - 1):
        issue_d(jnp.int32(k))

    xv = x_ref[...]
    logits = jax.lax.dot_general(
        xv, gatew_ref[...], (((1,), (1,)), ((), ())),
        preferred_element_type=jnp.float32)
    mx = jnp.max(logits, axis=-1, keepdims=True)
    p = jnp.exp(logits - mx)
    p = p / jnp.sum(p, axis=-1, keepdims=True)
    lanes = jax.lax.broadcasted_iota(jnp.int32, (T, E), 1)
    work = p
    selected = jnp.zeros((T, E), dtype=jnp.bool_)
    for _ in range(TOPK):
        idx = jnp.argmax(work, axis=-1).reshape(T, 1)
        oh = lanes == idx
        selected = jnp.logical_or(selected, oh)
        work = jnp.where(oh, -jnp.inf, work)
    psel = jnp.where(selected, p, 0.0)
    comb_ref[...] = psel / jnp.sum(psel, axis=-1, keepdims=True)
    sg_ref[...] = jax.lax.logistic(jax.lax.dot_general(
        xv, srw_ref[...], (((1,), (1,)), ((), ())),
        preferred_element_type=jnp.float32))
    xt_ref[...] = xv.astype(jnp.bfloat16).T
    out_ref[...] = jnp.zeros_like(out_ref)

    def body(v, _):
        xt = xt_ref[...]
        lanes_v = jax.lax.broadcasted_iota(jnp.int32, (T, E), 1)
        c_col = jnp.sum(jnp.where(lanes_v == v, comb_ref[...], 0.0),
                        axis=-1, keepdims=True)
        c_col = c_col + jnp.where(v >= E, sg_ref[...], 0.0)

        for m in range(NGU):
            j = v * NGU + m
            issue_gu(j + RW - 1)
            slot = j % RW
            pltpu.make_async_copy(
                ring_gu.at[slot], ring_gu.at[slot], sem_gu.at[slot]).wait()
            w = ring_gu[pl.ds(slot, 1)].reshape(MGU, D).astype(jnp.bfloat16)
            r = jax.lax.dot_general(
                w, xt, (((1,), (0,)), ((), ())),
                preferred_element_type=jnp.float32)
            row = (m % 2) * MGU
            if m < 2:
                gt_ref[row:row + MGU, :] = r
            else:
                g = gt_ref[row:row + MGU, :]
                gut_ref[row:row + MGU, :] = (
                    (g * jax.lax.logistic(g)) * r).astype(jnp.bfloat16)

        gut = gut_ref[...]
        for m in range(NDD):
            k = v * NDD + m
            issue_d(k + RD - 1)
            slot = k % RD
            pltpu.make_async_copy(
                ring_d.at[slot], ring_d.at[slot], sem_d.at[slot]).wait()
            dwc = ring_d[pl.ds(slot, 1)].reshape(MDD, H).astype(jnp.bfloat16)
            y = jax.lax.dot_general(
                gut, dwc, (((0,), (1,)), ((), ())),
                preferred_element_type=jnp.float32)
            out_ref[:, m * MDD:(m + 1) * MDD] += c_col * y
        return 0

    jax.lax.fori_loop(0, NV, body, 0)


def kernel(x, gate_w, expert_gate_w, expert_up_w, expert_down_w,
           shared_router_w, shared_gate_proj_w, shared_up_w, shared_down_w):
    b, l, d = x.shape
    xf = x.reshape(-1, d)

    out = pl.pallas_call(
        _mega_kernel,
        in_specs=[
            pl.BlockSpec(memory_space=pltpu.VMEM),
            pl.BlockSpec(memory_space=pltpu.VMEM),
            pl.BlockSpec(memory_space=pltpu.VMEM),
            pl.BlockSpec(memory_space=pl.ANY),
            pl.BlockSpec(memory_space=pl.ANY),
            pl.BlockSpec(memory_space=pl.ANY),
            pl.BlockSpec(memory_space=pl.ANY),
            pl.BlockSpec(memory_space=pl.ANY),
            pl.BlockSpec(memory_space=pl.ANY),
        ],
        out_specs=pl.BlockSpec(memory_space=pltpu.VMEM),
        out_shape=jax.ShapeDtypeStruct((T, D), jnp.float32),
        scratch_shapes=[
            pltpu.VMEM((RW, MGU, D), jnp.float32),
            pltpu.VMEM((RD, MDD, H), jnp.float32),
            pltpu.VMEM((D, T), jnp.bfloat16),
            pltpu.VMEM((H, T), jnp.float32),
            pltpu.VMEM((H, T), jnp.bfloat16),
            pltpu.VMEM((T, E), jnp.float32),
            pltpu.VMEM((T, 1), jnp.float32),
            pltpu.SemaphoreType.DMA((RW,)),
            pltpu.SemaphoreType.DMA((RD,)),
        ],
    )(xf, gate_w, shared_router_w,
      expert_gate_w, expert_up_w, expert_down_w.reshape(E * NDD, MDD, H),
      shared_gate_proj_w, shared_up_w, shared_down_w)

    return out.astype(x.dtype).reshape(b, l, d)


# speedup vs baseline: 1.0007x; 1.0007x over previous
import jax
import jax.numpy as jnp
from jax.experimental import pallas as pl
from jax.experimental.pallas import tpu as pltpu

D = 2048
E = 64
TOPK = 6
H = 1408
HS = 2816
T = 64

MGU = 704
NGU = 4
MDD = 1024
NDD = 2
NV = E + 2
RW = 4
RD = 3
JGU = NV * NGU
JD = NV * NDD
MGH = MGU // 2
MDH = MDD // 2


def _mega_kernel(x_ref, gatew_ref, srw_ref,
                 gw_hbm, uw_hbm, dw_hbm, sgw_hbm, suw_hbm, sdw_hbm,
                 out_ref,
                 ring_gu, ring_d, xt_ref, gt_ref, gut_ref, comb_ref, sg_ref,
                 sem_gu, sem_d):

    def issue_gu(j):
        @pl.when(j < JGU)
        def _():
            v = j // NGU
            m = j % NGU
            slot = j % RW
            row = (m % 2) * MGU

            def _expert():
                def _gate():
                    for h in range(2):
                        pltpu.make_async_copy(
                            gw_hbm.at[v, pl.ds(row + h * MGH, MGH), :],
                            ring_gu.at[slot, pl.ds(h * MGH, MGH), :],
                            sem_gu.at[slot, h]).start()

                def _up():
                    for h in range(2):
                        pltpu.make_async_copy(
                            uw_hbm.at[v, pl.ds(row + h * MGH, MGH), :],
                            ring_gu.at[slot, pl.ds(h * MGH, MGH), :],
                            sem_gu.at[slot, h]).start()
                jax.lax.cond(m < 2, _gate, _up)

            def _shared():
                srow = (v - E) * H + row

                def _gate():
                    for h in range(2):
                        pltpu.make_async_copy(
                            sgw_hbm.at[pl.ds(srow + h * MGH, MGH), :],
                            ring_gu.at[slot, pl.ds(h * MGH, MGH), :],
                            sem_gu.at[slot, h]).start()

                def _up():
                    for h in range(2):
                        pltpu.make_async_copy(
                            suw_hbm.at[pl.ds(srow + h * MGH, MGH), :],
                            ring_gu.at[slot, pl.ds(h * MGH, MGH), :],
                            sem_gu.at[slot, h]).start()
                jax.lax.cond(m < 2, _gate, _up)

            jax.lax.cond(v < E, _expert, _shared)

    def issue_d(k):
        @pl.when(k < JD)
        def _():
            v = k // NDD
            m = k % NDD
            slot = k % RD

            def _expert():
                for h in range(2):
                    pltpu.make_async_copy(
                        dw_hbm.at[v * NDD + m, pl.ds(h * MDH, MDH), :],
                        ring_d.at[slot, pl.ds(h * MDH, MDH), :],
                        sem_d.at[slot, h]).start()

            def _shared():
                for h in range(2):
                    pltpu.make_async_copy(
                        sdw_hbm.at[pl.ds(m * MDD + h * MDH, MDH),
                                   pl.ds((v - E) * H, H)],
                        ring_d.at[slot, pl.ds(h * MDH, MDH), :],
                        sem_d.at[slot, h]).start()

            jax.lax.cond(v < E, _expert, _shared)

    for j in range(RW - 1):
        issue_gu(jnp.int32(j))
    for k in range(RD - 1):
        issue_d(jnp.int32(k))

    xv = x_ref[...]
    logits = jax.lax.dot_general(
        xv, gatew_ref[...], (((1,), (1,)), ((), ())),
        preferred_element_type=jnp.float32)
    mx = jnp.max(logits, axis=-1, keepdims=True)
    p = jnp.exp(logits - mx)
    p = p / jnp.sum(p, axis=-1, keepdims=True)
    lanes = jax.lax.broadcasted_iota(jnp.int32, (T, E), 1)
    work = p
    selected = jnp.zeros((T, E), dtype=jnp.bool_)
    for _ in range(TOPK):
        idx = jnp.argmax(work, axis=-1).reshape(T, 1)
        oh = lanes == idx
        selected = jnp.logical_or(selected, oh)
        work = jnp.where(oh, -jnp.inf, work)
    psel = jnp.where(selected, p, 0.0)
    comb_ref[...] = psel / jnp.sum(psel, axis=-1, keepdims=True)
    sg_ref[...] = jax.lax.logistic(jax.lax.dot_general(
        xv, srw_ref[...], (((1,), (1,)), ((), ())),
        preferred_element_type=jnp.float32))
    xt_ref[...] = xv.astype(jnp.bfloat16).T
    out_ref[...] = jnp.zeros_like(out_ref)

    def body(v, _):
        xt = xt_ref[...]
        lanes_v = jax.lax.broadcasted_iota(jnp.int32, (T, E), 1)
        c_col = jnp.sum(jnp.where(lanes_v == v, comb_ref[...], 0.0),
                        axis=-1, keepdims=True)
        c_col = c_col + jnp.where(v >= E, sg_ref[...], 0.0)

        for m in range(NGU):
            j = v * NGU + m
            issue_gu(j + RW - 1)
            slot = j % RW
            for h in range(2):
                pltpu.make_async_copy(
                    ring_gu.at[slot, pl.ds(h * MGH, MGH), :],
                    ring_gu.at[slot, pl.ds(h * MGH, MGH), :],
                    sem_gu.at[slot, h]).wait()
            w = ring_gu[pl.ds(slot, 1)].reshape(MGU, D).astype(jnp.bfloat16)
            r = jax.lax.dot_general(
                w, xt, (((1,), (0,)), ((), ())),
                preferred_element_type=jnp.float32)
            row = (m % 2) * MGU
            if m < 2:
                gt_ref[row:row + MGU, :] = r
            else:
                g = gt_ref[row:row + MGU, :]
                gut_ref[row:row + MGU, :] = (
                    (g * jax.lax.logistic(g)) * r).astype(jnp.bfloat16)

        gut = gut_ref[...]
        for m in range(NDD):
            k = v * NDD + m
            issue_d(k + RD - 1)
            slot = k % RD
            for h in range(2):
                pltpu.make_async_copy(
                    ring_d.at[slot, pl.ds(h * MDH, MDH), :],
                    ring_d.at[slot, pl.ds(h * MDH, MDH), :],
                    sem_d.at[slot, h]).wait()
            dwc = ring_d[pl.ds(slot, 1)].reshape(MDD, H).astype(jnp.bfloat16)
            y = jax.lax.dot_general(
                gut, dwc, (((0,), (1,)), ((), ())),
                preferred_element_type=jnp.float32)
            out_ref[:, m * MDD:(m + 1) * MDD] += c_col * y
        return 0

    jax.lax.fori_loop(0, NV, body, 0)


def kernel(x, gate_w, expert_gate_w, expert_up_w, expert_down_w,
           shared_router_w, shared_gate_proj_w, shared_up_w, shared_down_w):
    b, l, d = x.shape
    xf = x.reshape(-1, d)

    out = pl.pallas_call(
        _mega_kernel,
        in_specs=[
            pl.BlockSpec(memory_space=pltpu.VMEM),
            pl.BlockSpec(memory_space=pltpu.VMEM),
            pl.BlockSpec(memory_space=pltpu.VMEM),
            pl.BlockSpec(memory_space=pl.ANY),
            pl.BlockSpec(memory_space=pl.ANY),
            pl.BlockSpec(memory_space=pl.ANY),
            pl.BlockSpec(memory_space=pl.ANY),
            pl.BlockSpec(memory_space=pl.ANY),
            pl.BlockSpec(memory_space=pl.ANY),
        ],
        out_specs=pl.BlockSpec(memory_space=pltpu.VMEM),
        out_shape=jax.ShapeDtypeStruct((T, D), jnp.float32),
        scratch_shapes=[
            pltpu.VMEM((RW, MGU, D), jnp.float32),
            pltpu.VMEM((RD, MDD, H), jnp.float32),
            pltpu.VMEM((D, T), jnp.bfloat16),
            pltpu.VMEM((H, T), jnp.float32),
            pltpu.VMEM((H, T), jnp.bfloat16),
            pltpu.VMEM((T, E), jnp.float32),
            pltpu.VMEM((T, 1), jnp.float32),
            pltpu.SemaphoreType.DMA((RW, 2)),
            pltpu.SemaphoreType.DMA((RD, 2)),
        ],
    )(xf, gate_w, shared_router_w,
      expert_gate_w, expert_up_w, expert_down_w.reshape(E * NDD, MDD, H),
      shared_gate_proj_w, shared_up_w, shared_down_w)

    return out.astype(x.dtype).reshape(b, l, d)


# speedup vs baseline: 1.0015x; 1.0008x over previous
import jax
import jax.numpy as jnp
from jax.experimental import pallas as pl
from jax.experimental.pallas import tpu as pltpu

D = 2048
E = 64
TOPK = 6
H = 1408
HS = 2816
T = 64

MGU = 704
NGU = 4
MDD = 1024
NDD = 2
NV = E + 2
RW = 4
RD = 3
JGU = NV * NGU
JD = NV * NDD


def _mega_kernel(x_ref, gatew_ref, srw_ref,
                 gw_hbm, uw_hbm, dw_hbm, sgw_hbm, suw_hbm, sdw_hbm,
                 out_ref,
                 ring_gu, ring_d, xt_ref, gt_ref, gut_ref, comb_ref, sg_ref,
                 sem_gu, sem_d):

    def issue_gu(j):
        @pl.when(j < JGU)
        def _():
            v = j // NGU
            m = j % NGU
            slot = j % RW
            row = (m % 2) * MGU

            def _expert():
                def _gate():
                    pltpu.make_async_copy(
                        gw_hbm.at[v, pl.ds(row, MGU), :],
                        ring_gu.at[slot], sem_gu.at[slot]).start()

                def _up():
                    pltpu.make_async_copy(
                        uw_hbm.at[v, pl.ds(row, MGU), :],
                        ring_gu.at[slot], sem_gu.at[slot]).start()
                jax.lax.cond(m < 2, _gate, _up)

            def _shared():
                srow = (v - E) * H + row

                def _gate():
                    pltpu.make_async_copy(
                        sgw_hbm.at[pl.ds(srow, MGU), :],
                        ring_gu.at[slot], sem_gu.at[slot]).start()

                def _up():
                    pltpu.make_async_copy(
                        suw_hbm.at[pl.ds(srow, MGU), :],
                        ring_gu.at[slot], sem_gu.at[slot]).start()
                jax.lax.cond(m < 2, _gate, _up)

            jax.lax.cond(v < E, _expert, _shared)

    def issue_d(k):
        @pl.when(k < JD)
        def _():
            v = k // NDD
            m = k % NDD
            slot = k % RD

            def _expert():
                pltpu.make_async_copy(
                    dw_hbm.at[v * NDD + m],
                    ring_d.at[slot], sem_d.at[slot]).start()

            def _shared():
                pltpu.make_async_copy(
                    sdw_hbm.at[pl.ds(m * MDD, MDD), pl.ds((v - E) * H, H)],
                    ring_d.at[slot], sem_d.at[slot]).start()

            jax.lax.cond(v < E, _expert, _shared)

    for j in range(RW - 1):
        issue_gu(jnp.int32(j))
    for k in range(RD - 1):
        issue_d(jnp.int32(k))

    xv = x_ref[...]
    logits = jax.lax.dot_general(
        xv, gatew_ref[...], (((1,), (1,)), ((), ())),
        preferred_element_type=jnp.float32)
    mx = jnp.max(logits, axis=-1, keepdims=True)
    p = jnp.exp(logits - mx)
    p = p / jnp.sum(p, axis=-1, keepdims=True)
    lanes = jax.lax.broadcasted_iota(jnp.int32, (T, E), 1)
    work = p
    selected = jnp.zeros((T, E), dtype=jnp.bool_)
    for _ in range(TOPK):
        idx = jnp.argmax(work, axis=-1).reshape(T, 1)
        oh = lanes == idx
        selected = jnp.logical_or(selected, oh)
        work = jnp.where(oh, -jnp.inf, work)
    psel = jnp.where(selected, p, 0.0)
    comb_ref[...] = psel / jnp.sum(psel, axis=-1, keepdims=True)
    sg_ref[...] = jax.lax.logistic(jax.lax.dot_general(
        xv, srw_ref[...], (((1,), (1,)), ((), ())),
        preferred_element_type=jnp.float32))
    xt_ref[...] = xv.astype(jnp.bfloat16).T
    out_ref[...] = jnp.zeros_like(out_ref)

    def body(v, _):
        xt = xt_ref[...]
        lanes_v = jax.lax.broadcasted_iota(jnp.int32, (T, E), 1)
        c_col = jnp.sum(jnp.where(lanes_v == v, comb_ref[...], 0.0),
                        axis=-1, keepdims=True)
        c_col = c_col + jnp.where(v >= E, sg_ref[...], 0.0)

        for m in range(NGU):
            j = v * NGU + m
            issue_gu(j + RW - 1)
            slot = j % RW
            pltpu.make_async_copy(
                ring_gu.at[slot], ring_gu.at[slot], sem_gu.at[slot]).wait()
            w = ring_gu[pl.ds(slot, 1)].reshape(MGU, D).astype(jnp.bfloat16)
            r = jax.lax.dot_general(
                w, xt, (((1,), (0,)), ((), ())),
                preferred_element_type=jnp.float32)
            row = (m % 2) * MGU
            if m < 2:
                gt_ref[row:row + MGU, :] = r
            else:
                g = gt_ref[row:row + MGU, :]
                gut_ref[row:row + MGU, :] = (
                    (g * jax.lax.logistic(g)) * r).astype(jnp.bfloat16)

        gut = gut_ref[...]
        for m in range(NDD):
            k = v * NDD + m
            issue_d(k + RD - 1)
            slot = k % RD
            pltpu.make_async_copy(
                ring_d.at[slot], ring_d.at[slot], sem_d.at[slot]).wait()
            dwc = ring_d[pl.ds(slot, 1)].reshape(MDD, H).astype(jnp.bfloat16)
            y = jax.lax.dot_general(
                gut, dwc, (((0,), (1,)), ((), ())),
                preferred_element_type=jnp.float32)
            out_ref[:, m * MDD:(m + 1) * MDD] += c_col * y
        return 0

    jax.lax.fori_loop(0, NV, body, 0)


def kernel(x, gate_w, expert_gate_w, expert_up_w, expert_down_w,
           shared_router_w, shared_gate_proj_w, shared_up_w, shared_down_w):
    b, l, d = x.shape
    xf = x.reshape(-1, d)

    out = pl.pallas_call(
        _mega_kernel,
        in_specs=[
            pl.BlockSpec(memory_space=pltpu.VMEM),
            pl.BlockSpec(memory_space=pltpu.VMEM),
            pl.BlockSpec(memory_space=pltpu.VMEM),
            pl.BlockSpec(memory_space=pl.ANY),
            pl.BlockSpec(memory_space=pl.ANY),
            pl.BlockSpec(memory_space=pl.ANY),
            pl.BlockSpec(memory_space=pl.ANY),
            pl.BlockSpec(memory_space=pl.ANY),
            pl.BlockSpec(memory_space=pl.ANY),
        ],
        out_specs=pl.BlockSpec(memory_space=pltpu.VMEM),
        out_shape=jax.ShapeDtypeStruct((T, D), jnp.float32),
        scratch_shapes=[
            pltpu.VMEM((RW, MGU, D), jnp.float32),
            pltpu.VMEM((RD, MDD, H), jnp.float32),
            pltpu.VMEM((D, T), jnp.bfloat16),
            pltpu.VMEM((H, T), jnp.float32),
            pltpu.VMEM((H, T), jnp.bfloat16),
            pltpu.VMEM((T, E), jnp.float32),
            pltpu.VMEM((T, 1), jnp.float32),
            pltpu.SemaphoreType.DMA((RW,)),
            pltpu.SemaphoreType.DMA((RD,)),
        ],
    )(xf, gate_w, shared_router_w,
      expert_gate_w, expert_up_w, expert_down_w.reshape(E * NDD, MDD, H),
      shared_gate_proj_w, shared_up_w, shared_down_w)

    return out.astype(x.dtype).reshape(b, l, d)
